# baseline (speedup 1.0000x reference)
import functools

import jax
import jax.numpy as jnp
from jax import lax
from jax.experimental import pallas as pl
from jax.experimental.pallas import tpu as pltpu
from jax.experimental.pallas import tpu_sc as plsc

_N = 10000
_E = 160000
_NIN = 128
_NOUT = 128
_CB = 32
_K = 27

_NC = 2
_NS = 16
_L = 16

_EPAD = 163840
_EALLOC = 167936
_CHUNK = 128
_NCH0 = 56
_NCH1 = 24
_EPT0 = _NCH0 * _CHUNK
_EPT1 = _NCH1 * _CHUNK
_NBUF = 8
_NACC = 10240
_JUNK = 10200
_RPT = _NACC // _NS
_ZR = 64
_NT = 7


def _tc1_body(x_ref, w1_ref, b1_ref, w2_ref, ht_ref):
    h = jnp.dot(x_ref[...], w1_ref[...], preferred_element_type=jnp.float32)
    h = jnp.maximum(h + b1_ref[...], 0.0)
    for t in range(_NT):
        ht_ref[t] = jnp.dot(h, w2_ref[:, pl.ds(t * 128, 128)],
                            preferred_element_type=jnp.float32)


def _tc2_body(p_ref, x_ref, b2_ref, w3_ref, b3_ref, o_ref):
    for q in range(4):
        qs = pl.ds(q * _CB, _CB)
        h2 = jnp.maximum(p_ref[0, :, qs] + p_ref[1, :, qs] + b2_ref[...], 0.0)
        h3 = jnp.dot(h2, w3_ref[...], preferred_element_type=jnp.float32)
        qo = pl.ds(q * _NOUT, _NOUT)
        o_ref[:, qo] = jnp.maximum(
            (h3 + b3_ref[...] + x_ref[:, qo]) * 0.5, 0.0)


def _sc_body(ht_hbm, src_hbm, kidx_hbm, dst_hbm, out_hbm,
             srcbuf, kidxbuf, gidx, dstbuf, rows, zbuf, acc,
             sg0, sg1, sg2, sg3, sg4, sg5, sg6, sg7, lsem):
    sems = (sg0, sg1, sg2, sg3, sg4, sg5, sg6, sg7)
    c = lax.axis_index("c")
    s = lax.axis_index("s")
    nch = jnp.where(c == 0, _NCH0, _NCH1)
    ebase = jnp.where(c == 0, s * _EPT0, _NS * _EPT0 + s * _EPT1)

    pltpu.async_copy(src_hbm.at[pl.ds(ebase, _EPT0)], srcbuf, lsem)
    pltpu.async_copy(kidx_hbm.at[pl.ds(ebase, _EPT0)], kidxbuf, lsem)
    pltpu.async_copy(dst_hbm.at[pl.ds(ebase // _CHUNK, _NCH0)], dstbuf, lsem)

    z16 = jnp.zeros((_L,), jnp.float32)

    def zero_zbuf(r, carry):
        zbuf[r, pl.ds(0, _L)] = z16
        zbuf[r, pl.ds(_L, _L)] = z16
        return carry
    lax.fori_loop(0, _ZR, zero_zbuf, 0)

    pltpu.make_async_copy(src_hbm.at[pl.ds(0, _EPT0)], srcbuf, lsem).wait()
    pltpu.make_async_copy(src_hbm.at[pl.ds(0, _EPT0)], kidxbuf, lsem).wait()
    pltpu.make_async_copy(dst_hbm.at[pl.ds(0, _NCH0)], dstbuf, lsem).wait()

    for t in range(_RPT // _ZR):
        pltpu.async_copy(zbuf, acc.at[pl.ds(s * _RPT + t * _ZR, _ZR)], lsem)

    def build_idx(i, carry):
        o = i * _L
        kv = kidxbuf[pl.ds(o, _L)]
        sv = srcbuf[pl.ds(o, _L)]
        gidx[pl.ds(o, _L)] = (
            lax.shift_right_logical(kv, 2) * (4 * _N)
            + sv * 4 + lax.bitwise_and(kv, 3))
        return carry
    lax.fori_loop(0, _EPT0 // _L, build_idx, 0)

    def gather(j, b):
        pltpu.async_copy(ht_hbm.at[gidx.at[pl.ds(j * _CHUNK, _CHUNK)]],
                         rows.at[b], sems[b])

    def wait_gather(b):
        pltpu.make_async_copy(ht_hbm.at[pl.ds(0, _CHUNK)], rows.at[b],
                              sems[b]).wait()

    for b in range(_NBUF):
        gather(b, b)

    for t in range(_RPT // _ZR):
        pltpu.make_async_copy(ht_hbm.at[pl.ds(0, _ZR)], zbuf, lsem).wait()
    plsc.subcore_barrier()

    def chunk_round(jj, carry):
        j = jj * _NBUF
        for b in range(_NBUF):
            wait_gather(b)
            pltpu.sync_copy(rows.at[b], acc.at[dstbuf.at[j + b]], add=True)

            @pl.when(j + b + _NBUF < nch)
            def _():
                gather(j + b + _NBUF, b)
        return carry
    lax.fori_loop(0, nch // _NBUF, chunk_round, 0)
    plsc.subcore_barrier()

    pltpu.sync_copy(acc.at[pl.ds(s * _RPT, _RPT)],
                    out_hbm.at[pl.ds(c * _NACC + s * _RPT, _RPT)])


@jax.jit
def kernel(x, W1, b1, W2, b2, W3, b3, edge_src, edge_dst, edge_kidx):
    npad = _EALLOC - _E
    src_p = jnp.concatenate([edge_src, jnp.zeros((npad,), jnp.int32)])
    kidx_p = jnp.concatenate([edge_kidx, jnp.zeros((npad,), jnp.int32)])
    dst_p = jnp.concatenate(
        [edge_dst, jnp.full((npad,), _JUNK, jnp.int32)]).reshape(-1, _CHUNK)

    bn1 = 512
    grid1 = pl.cdiv(_N, bn1)
    w2all = jnp.transpose(W2, (1, 0, 2)).reshape(_CB, _K * _CB)
    w2pad = jnp.concatenate(
        [w2all, jnp.zeros((_CB, _NT * 128 - _K * _CB), jnp.float32)], axis=1)
    ht = pl.pallas_call(
        _tc1_body,
        grid=(grid1,),
        in_specs=[
            pl.BlockSpec((bn1, _NIN), lambda i: (i, 0)),
            pl.BlockSpec((_NIN, _CB), lambda i: (0, 0)),
            pl.BlockSpec((1, _CB), lambda i: (0, 0)),
            pl.BlockSpec((_CB, _NT * 128), lambda i: (0, 0)),
        ],
        out_specs=pl.BlockSpec((_NT, bn1, 128), lambda i: (0, i, 0)),
        out_shape=jax.ShapeDtypeStruct((_NT, _N, 128), jnp.float32),
    )(x, W1, b1.reshape(1, _CB), w2pad)
    ht_rows = ht.reshape(_NT * _N * 4, _CB)

    mesh = plsc.VectorSubcoreMesh(core_axis_name="c", subcore_axis_name="s")
    partial = pl.kernel(
        _sc_body,
        out_type=jax.ShapeDtypeStruct((_NC * _NACC, _CB), jnp.float32),
        mesh=mesh,
        compiler_params=pltpu.CompilerParams(use_tc_tiling_on_sc=False),
        scratch_types=[
            pltpu.VMEM((_EPT0,), jnp.int32),
            pltpu.VMEM((_EPT0,), jnp.int32),
            pltpu.VMEM((_EPT0,), jnp.int32),
            pltpu.VMEM((_NCH0, _CHUNK), jnp.int32),
            pltpu.VMEM((_NBUF, _CHUNK, _CB), jnp.float32),
            pltpu.VMEM((_ZR, _CB), jnp.float32),
            pltpu.VMEM_SHARED((_NACC, _CB), jnp.float32),
        ] + [pltpu.SemaphoreType.DMA] * (_NBUF + 1),
    )(ht_rows, src_p, kidx_p, dst_p)
    psum = partial.reshape(_NC, _NACC // 4, 4 * _CB)
    x4 = x.reshape(_N // 4, 4 * _NIN)

    bn2 = 128
    grid2 = pl.cdiv(_N // 4, bn2)
    out4 = pl.pallas_call(
        _tc2_body,
        grid=(grid2,),
        in_specs=[
            pl.BlockSpec((_NC, bn2, 4 * _CB), lambda i: (0, i, 0)),
            pl.BlockSpec((bn2, 4 * _NIN), lambda i: (i, 0)),
            pl.BlockSpec((1, _CB), lambda i: (0, 0)),
            pl.BlockSpec((_CB, _NOUT), lambda i: (0, 0)),
            pl.BlockSpec((1, _NOUT), lambda i: (0, 0)),
        ],
        out_specs=pl.BlockSpec((bn2, 4 * _NOUT), lambda i: (i, 0)),
        out_shape=jax.ShapeDtypeStruct((_N // 4, 4 * _NOUT), jnp.float32),
    )(psum, x4, b2.reshape(1, _CB), W3, b3.reshape(1, _NOUT))
    return out4.reshape(_N, _NOUT)


# speedup vs baseline: 1.0065x; 1.0065x over previous
import functools

import jax
import jax.numpy as jnp
from jax import lax
from jax.experimental import pallas as pl
from jax.experimental.pallas import tpu as pltpu
from jax.experimental.pallas import tpu_sc as plsc

_N = 10000
_E = 160000
_NIN = 128
_NOUT = 128
_CB = 32
_K = 27

_NC = 2
_NS = 16
_L = 16

_EPAD = 163840
_CHUNK = 128
_NCH = 80
_EPT = _NCH * _CHUNK
_NBUF = 8
_NACC = 10240
_JUNK = 10200
_RPT = _NACC // _NS
_ZR = 64
_NT = 7


def _tc1_body(x_ref, w1_ref, b1_ref, w2_ref, ht_ref):
    h = jnp.dot(x_ref[...], w1_ref[...], preferred_element_type=jnp.float32)
    h = jnp.maximum(h + b1_ref[...], 0.0)
    for t in range(_NT):
        ht_ref[t] = jnp.dot(h, w2_ref[:, pl.ds(t * 128, 128)],
                            preferred_element_type=jnp.float32)


def _tc2_body(p_ref, x_ref, b2_ref, w3_ref, b3_ref, o_ref):
    for q in range(4):
        qs = pl.ds(q * _CB, _CB)
        h2 = jnp.maximum(p_ref[:, qs] + b2_ref[...], 0.0)
        h3 = jnp.dot(h2, w3_ref[...], preferred_element_type=jnp.float32)
        qo = pl.ds(q * _NOUT, _NOUT)
        o_ref[:, qo] = jnp.maximum(
            (h3 + b3_ref[...] + x_ref[:, qo]) * 0.5, 0.0)


def _sc_body(ht_hbm, src_hbm, kidx_hbm, dst_hbm, out_hbm,
             srcbuf, kidxbuf, gidx, dstbuf, rows, zbuf, acc,
             sg0, sg1, sg2, sg3, sg4, sg5, sg6, sg7, lsem):
    sems = (sg0, sg1, sg2, sg3, sg4, sg5, sg6, sg7)
    c = lax.axis_index("c")
    s = lax.axis_index("s")
    ebase = s * _EPT

    def gather(j, b):
        pltpu.async_copy(ht_hbm.at[gidx.at[pl.ds(j * _CHUNK, _CHUNK)]],
                         rows.at[b], sems[b])

    def wait_gather(b):
        pltpu.make_async_copy(ht_hbm.at[pl.ds(0, _CHUNK)], rows.at[b],
                              sems[b]).wait()

    @pl.when(c == 0)
    def _prologue():
        pltpu.async_copy(src_hbm.at[pl.ds(ebase, _EPT)], srcbuf, lsem)
        pltpu.async_copy(kidx_hbm.at[pl.ds(ebase, _EPT)], kidxbuf, lsem)
        pltpu.async_copy(dst_hbm.at[pl.ds(s * _NCH, _NCH)], dstbuf, lsem)

        z16 = jnp.zeros((_L,), jnp.float32)

        def zero_zbuf(r, carry):
            zbuf[r, pl.ds(0, _L)] = z16
            zbuf[r, pl.ds(_L, _L)] = z16
            return carry
        lax.fori_loop(0, _ZR, zero_zbuf, 0)

        pltpu.make_async_copy(src_hbm.at[pl.ds(0, _EPT)], srcbuf, lsem).wait()
        pltpu.make_async_copy(src_hbm.at[pl.ds(0, _EPT)], kidxbuf, lsem).wait()
        pltpu.make_async_copy(dst_hbm.at[pl.ds(0, _NCH)], dstbuf, lsem).wait()

        for t in range(_RPT // _ZR):
            pltpu.async_copy(zbuf, acc.at[pl.ds(s * _RPT + t * _ZR, _ZR)],
                             lsem)

        def build_idx(i, carry):
            o = i * _L
            kv = kidxbuf[pl.ds(o, _L)]
            sv = srcbuf[pl.ds(o, _L)]
            gidx[pl.ds(o, _L)] = (
                lax.shift_right_logical(kv, 2) * (4 * _N)
                + sv * 4 + lax.bitwise_and(kv, 3))
            return carry
        lax.fori_loop(0, _EPT // _L, build_idx, 0)

        for b in range(_NBUF):
            gather(b, b)

        for t in range(_RPT // _ZR):
            pltpu.make_async_copy(ht_hbm.at[pl.ds(0, _ZR)], zbuf, lsem).wait()

    plsc.subcore_barrier()

    @pl.when(c == 0)
    def _main():
        def chunk_round(jj, carry):
            j = jj * _NBUF
            for b in range(_NBUF):
                wait_gather(b)
                pltpu.sync_copy(rows.at[b], acc.at[dstbuf.at[j + b]],
                                add=True)

                @pl.when(j + b + _NBUF < _NCH)
                def _():
                    gather(j + b + _NBUF, b)
            return carry
        lax.fori_loop(0, _NCH // _NBUF, chunk_round, 0)

    plsc.subcore_barrier()

    @pl.when(c == 0)
    def _epilogue():
        pltpu.sync_copy(acc.at[pl.ds(s * _RPT, _RPT)],
                        out_hbm.at[pl.ds(s * _RPT, _RPT)])


@jax.jit
def kernel(x, W1, b1, W2, b2, W3, b3, edge_src, edge_dst, edge_kidx):
    npad = _EPAD - _E
    src_p = jnp.concatenate([edge_src, jnp.zeros((npad,), jnp.int32)])
    kidx_p = jnp.concatenate([edge_kidx, jnp.zeros((npad,), jnp.int32)])
    dst_p = jnp.concatenate(
        [edge_dst, jnp.full((npad,), _JUNK, jnp.int32)]).reshape(-1, _CHUNK)

    bn1 = 512
    grid1 = pl.cdiv(_N, bn1)
    w2all = jnp.transpose(W2, (1, 0, 2)).reshape(_CB, _K * _CB)
    w2pad = jnp.concatenate(
        [w2all, jnp.zeros((_CB, _NT * 128 - _K * _CB), jnp.float32)], axis=1)
    ht = pl.pallas_call(
        _tc1_body,
        grid=(grid1,),
        in_specs=[
            pl.BlockSpec((bn1, _NIN), lambda i: (i, 0)),
            pl.BlockSpec((_NIN, _CB), lambda i: (0, 0)),
            pl.BlockSpec((1, _CB), lambda i: (0, 0)),
            pl.BlockSpec((_CB, _NT * 128), lambda i: (0, 0)),
        ],
        out_specs=pl.BlockSpec((_NT, bn1, 128), lambda i: (0, i, 0)),
        out_shape=jax.ShapeDtypeStruct((_NT, _N, 128), jnp.float32),
    )(x, W1, b1.reshape(1, _CB), w2pad)
    ht_rows = ht.reshape(_NT * _N * 4, _CB)

    mesh = plsc.VectorSubcoreMesh(core_axis_name="c", subcore_axis_name="s")
    partial = pl.kernel(
        _sc_body,
        out_type=jax.ShapeDtypeStruct((_NACC, _CB), jnp.float32),
        mesh=mesh,
        compiler_params=pltpu.CompilerParams(use_tc_tiling_on_sc=False),
        scratch_types=[
            pltpu.VMEM((_EPT,), jnp.int32),
            pltpu.VMEM((_EPT,), jnp.int32),
            pltpu.VMEM((_EPT,), jnp.int32),
            pltpu.VMEM((_NCH, _CHUNK), jnp.int32),
            pltpu.VMEM((_NBUF, _CHUNK, _CB), jnp.float32),
            pltpu.VMEM((_ZR, _CB), jnp.float32),
            pltpu.VMEM_SHARED((_NACC, _CB), jnp.float32),
        ] + [pltpu.SemaphoreType.DMA] * (_NBUF + 1),
    )(ht_rows, src_p, kidx_p, dst_p)
    psum = partial.reshape(_NACC // 4, 4 * _CB)
    x4 = x.reshape(_N // 4, 4 * _NIN)

    bn2 = 128
    grid2 = pl.cdiv(_N // 4, bn2)
    out4 = pl.pallas_call(
        _tc2_body,
        grid=(grid2,),
        in_specs=[
            pl.BlockSpec((bn2, 4 * _CB), lambda i: (i, 0)),
            pl.BlockSpec((bn2, 4 * _NIN), lambda i: (i, 0)),
            pl.BlockSpec((1, _CB), lambda i: (0, 0)),
            pl.BlockSpec((_CB, _NOUT), lambda i: (0, 0)),
            pl.BlockSpec((1, _NOUT), lambda i: (0, 0)),
        ],
        out_specs=pl.BlockSpec((bn2, 4 * _NOUT), lambda i: (i, 0)),
        out_shape=jax.ShapeDtypeStruct((_N // 4, 4 * _NOUT), jnp.float32),
    )(psum, x4, b2.reshape(1, _CB), W3, b3.reshape(1, _NOUT))
    return out4.reshape(_N, _NOUT)


# speedup vs baseline: 1.4894x; 1.4799x over previous
import functools

import jax
import jax.numpy as jnp
from jax import lax
from jax.experimental import pallas as pl
from jax.experimental.pallas import tpu as pltpu
from jax.experimental.pallas import tpu_sc as plsc

_N = 10000
_E = 160000
_NIN = 128
_NOUT = 128
_CB = 32
_K = 27

_NC = 2
_NS = 16
_L = 16

_EPAD = 163840
_CHUNK = 128
_NCH = 40
_EPT = _NCH * _CHUNK
_NBUF = 8
_NACC = 10240
_JUNK = 10200
_RPT = _NACC // _NS
_ZR = 64
_NT = 7


def _tc1_body(x_ref, w1_ref, b1_ref, w2_ref, ht_ref):
    h = jnp.dot(x_ref[...], w1_ref[...], preferred_element_type=jnp.float32)
    h = jnp.maximum(h + b1_ref[...], 0.0)
    for t in range(_NT):
        ht_ref[t] = jnp.dot(h, w2_ref[:, pl.ds(t * 128, 128)],
                            preferred_element_type=jnp.float32)


def _tc2_body(p_ref, x_ref, b2_ref, w3_ref, b3_ref, o_ref):
    for q in range(4):
        qs = pl.ds(q * _CB, _CB)
        h2 = jnp.maximum(p_ref[0, :, qs] + p_ref[1, :, qs] + b2_ref[...], 0.0)
        h3 = jnp.dot(h2, w3_ref[...], preferred_element_type=jnp.float32)
        qo = pl.ds(q * _NOUT, _NOUT)
        o_ref[:, qo] = jnp.maximum(
            (h3 + b3_ref[...] + x_ref[:, qo]) * 0.5, 0.0)


def _sc_body(ht_hbm, src_hbm, kidx_hbm, dst_hbm, out_hbm,
             srcbuf, kidxbuf, gidx, dstbuf, rows, zbuf, acc,
             sg0, sg1, sg2, sg3, sg4, sg5, sg6, sg7, lsem):
    sems = (sg0, sg1, sg2, sg3, sg4, sg5, sg6, sg7)
    c = lax.axis_index("c")
    s = lax.axis_index("s")
    wid = c * _NS + s
    ebase = wid * _EPT

    def gather(j, b):
        pltpu.async_copy(ht_hbm.at[gidx.at[pl.ds(j * _CHUNK, _CHUNK)]],
                         rows.at[b], sems[b])

    def wait_gather(b):
        pltpu.make_async_copy(ht_hbm.at[pl.ds(0, _CHUNK)], rows.at[b],
                              sems[b]).wait()

    pltpu.async_copy(src_hbm.at[pl.ds(ebase, _EPT)], srcbuf, lsem)
    pltpu.async_copy(kidx_hbm.at[pl.ds(ebase, _EPT)], kidxbuf, lsem)
    pltpu.async_copy(dst_hbm.at[pl.ds(wid * _NCH, _NCH)], dstbuf, lsem)

    z16 = jnp.zeros((_L,), jnp.float32)

    def zero_zbuf(r, carry):
        zbuf[r, pl.ds(0, _L)] = z16
        zbuf[r, pl.ds(_L, _L)] = z16
        return carry
    lax.fori_loop(0, _ZR, zero_zbuf, 0)

    pltpu.make_async_copy(src_hbm.at[pl.ds(0, _EPT)], srcbuf, lsem).wait()
    pltpu.make_async_copy(src_hbm.at[pl.ds(0, _EPT)], kidxbuf, lsem).wait()
    pltpu.make_async_copy(dst_hbm.at[pl.ds(0, _NCH)], dstbuf, lsem).wait()

    for t in range(_RPT // _ZR):
        pltpu.async_copy(zbuf, acc.at[pl.ds(s * _RPT + t * _ZR, _ZR)], lsem)

    def build_idx(i, carry):
        o = i * _L
        kv = kidxbuf[pl.ds(o, _L)]
        sv = srcbuf[pl.ds(o, _L)]
        gidx[pl.ds(o, _L)] = (
            lax.shift_right_logical(kv, 2) * (4 * _N)
            + sv * 4 + lax.bitwise_and(kv, 3))
        return carry
    lax.fori_loop(0, _EPT // _L, build_idx, 0)

    for b in range(_NBUF):
        gather(b, b)

    for t in range(_RPT // _ZR):
        pltpu.make_async_copy(ht_hbm.at[pl.ds(0, _ZR)], zbuf, lsem).wait()
    plsc.subcore_barrier()

    def chunk_round(jj, carry):
        j = jj * _NBUF
        for b in range(_NBUF):
            wait_gather(b)
            pltpu.sync_copy(rows.at[b], acc.at[dstbuf.at[j + b]], add=True)

            @pl.when(j + b + _NBUF < _NCH)
            def _():
                gather(j + b + _NBUF, b)
        return carry
    lax.fori_loop(0, _NCH // _NBUF, chunk_round, 0)
    plsc.subcore_barrier()

    pltpu.sync_copy(acc.at[pl.ds(s * _RPT, _RPT)],
                    out_hbm.at[pl.ds(c * _NACC + s * _RPT, _RPT)])


@jax.jit
def kernel(x, W1, b1, W2, b2, W3, b3, edge_src, edge_dst, edge_kidx):
    npad = _EPAD - _E
    spread = jnp.arange(npad, dtype=jnp.int32)
    src_p = jnp.concatenate([edge_src, spread % _N])
    kidx_p = jnp.concatenate([edge_kidx, jnp.zeros((npad,), jnp.int32)])
    dst_p = jnp.concatenate(
        [edge_dst, _N + spread % (_NACC - _N)]).reshape(-1, _CHUNK)

    bn1 = 512
    grid1 = pl.cdiv(_N, bn1)
    w2all = jnp.transpose(W2, (1, 0, 2)).reshape(_CB, _K * _CB)
    w2pad = jnp.concatenate(
        [w2all, jnp.zeros((_CB, _NT * 128 - _K * _CB), jnp.float32)], axis=1)
    ht = pl.pallas_call(
        _tc1_body,
        grid=(grid1,),
        in_specs=[
            pl.BlockSpec((bn1, _NIN), lambda i: (i, 0)),
            pl.BlockSpec((_NIN, _CB), lambda i: (0, 0)),
            pl.BlockSpec((1, _CB), lambda i: (0, 0)),
            pl.BlockSpec((_CB, _NT * 128), lambda i: (0, 0)),
        ],
        out_specs=pl.BlockSpec((_NT, bn1, 128), lambda i: (0, i, 0)),
        out_shape=jax.ShapeDtypeStruct((_NT, _N, 128), jnp.float32),
    )(x, W1, b1.reshape(1, _CB), w2pad)
    ht_rows = ht.reshape(_NT * _N * 4, _CB)

    mesh = plsc.VectorSubcoreMesh(core_axis_name="c", subcore_axis_name="s")
    partial = pl.kernel(
        _sc_body,
        out_type=jax.ShapeDtypeStruct((_NC * _NACC, _CB), jnp.float32),
        mesh=mesh,
        compiler_params=pltpu.CompilerParams(use_tc_tiling_on_sc=False),
        scratch_types=[
            pltpu.VMEM((_EPT,), jnp.int32),
            pltpu.VMEM((_EPT,), jnp.int32),
            pltpu.VMEM((_EPT,), jnp.int32),
            pltpu.VMEM((_NCH, _CHUNK), jnp.int32),
            pltpu.VMEM((_NBUF, _CHUNK, _CB), jnp.float32),
            pltpu.VMEM((_ZR, _CB), jnp.float32),
            pltpu.VMEM_SHARED((_NACC, _CB), jnp.float32),
        ] + [pltpu.SemaphoreType.DMA] * (_NBUF + 1),
    )(ht_rows, src_p, kidx_p, dst_p)
    psum = partial.reshape(_NC, _NACC // 4, 4 * _CB)
    x4 = x.reshape(_N // 4, 4 * _NIN)

    bn2 = 128
    grid2 = pl.cdiv(_N // 4, bn2)
    out4 = pl.pallas_call(
        _tc2_body,
        grid=(grid2,),
        in_specs=[
            pl.BlockSpec((_NC, bn2, 4 * _CB), lambda i: (0, i, 0)),
            pl.BlockSpec((bn2, 4 * _NIN), lambda i: (i, 0)),
            pl.BlockSpec((1, _CB), lambda i: (0, 0)),
            pl.BlockSpec((_CB, _NOUT), lambda i: (0, 0)),
            pl.BlockSpec((1, _NOUT), lambda i: (0, 0)),
        ],
        out_specs=pl.BlockSpec((bn2, 4 * _NOUT), lambda i: (i, 0)),
        out_shape=jax.ShapeDtypeStruct((_N // 4, 4 * _NOUT), jnp.float32),
    )(psum, x4, b2.reshape(1, _CB), W3, b3.reshape(1, _NOUT))
    return out4.reshape(_N, _NOUT)


# speedup vs baseline: 1.7677x; 1.1868x over previous
import functools

import jax
import jax.numpy as jnp
from jax import lax
from jax.experimental import pallas as pl
from jax.experimental.pallas import tpu as pltpu
from jax.experimental.pallas import tpu_sc as plsc

_N = 10000
_E = 160000
_NIN = 128
_NOUT = 128
_CB = 32
_K = 27

_NC = 2
_NS = 16
_L = 16

_EPAD = 163840
_CHUNK = 128
_NCH = 40
_EPT = _NCH * _CHUNK
_NBUF = 8
_NACC = 10240
_JUNK = 10200
_RPT = _NACC // _NS
_ZR = 64
_NT = 7


def _tc1_body(x_ref, w1_ref, b1_ref, w2_ref, ht_ref):
    h = jnp.dot(x_ref[...], w1_ref[...], preferred_element_type=jnp.float32)
    h = jnp.maximum(h + b1_ref[...], 0.0)
    for t in range(_NT):
        ht_ref[t] = jnp.dot(h, w2_ref[:, pl.ds(t * 128, 128)],
                            preferred_element_type=jnp.float32)


def _tc2_body(p_ref, x_ref, b2_ref, w3_ref, b3_ref, o_ref):
    bn = p_ref.shape[1]
    slabs = []
    for q in range(4):
        qs = pl.ds(q * _CB, _CB)
        h2 = jnp.maximum(p_ref[0, :, qs] + p_ref[1, :, qs] + b2_ref[...], 0.0)
        h3 = jnp.dot(h2, w3_ref[...], preferred_element_type=jnp.float32)
        qo = pl.ds(q * _NOUT, _NOUT)
        slabs.append(jnp.maximum(
            (h3 + b3_ref[...] + x_ref[:, qo]) * 0.5, 0.0))
    y = jnp.concatenate(slabs, axis=1)
    o_ref[...] = y.reshape(4 * bn, _NOUT)


def _sc_body(ht_hbm, src_hbm, kidx_hbm, dst_hbm, out_hbm,
             srcbuf, kidxbuf, gidx, dstbuf, rows, zbuf, acc,
             sg0, sg1, sg2, sg3, sg4, sg5, sg6, sg7, lsem):
    sems = (sg0, sg1, sg2, sg3, sg4, sg5, sg6, sg7)
    c = lax.axis_index("c")
    s = lax.axis_index("s")
    wid = c * _NS + s
    ebase = wid * _EPT

    def gather(j, b):
        pltpu.async_copy(ht_hbm.at[gidx.at[pl.ds(j * _CHUNK, _CHUNK)]],
                         rows.at[b], sems[b])

    def wait_gather(b):
        pltpu.make_async_copy(ht_hbm.at[pl.ds(0, _CHUNK)], rows.at[b],
                              sems[b]).wait()

    pltpu.async_copy(src_hbm.at[pl.ds(ebase, _EPT)], srcbuf, lsem)
    pltpu.async_copy(kidx_hbm.at[pl.ds(ebase, _EPT)], kidxbuf, lsem)
    pltpu.async_copy(dst_hbm.at[pl.ds(wid * _NCH, _NCH)], dstbuf, lsem)

    z16 = jnp.zeros((_L,), jnp.float32)

    def zero_zbuf(r, carry):
        zbuf[r, pl.ds(0, _L)] = z16
        zbuf[r, pl.ds(_L, _L)] = z16
        return carry
    lax.fori_loop(0, _ZR, zero_zbuf, 0)

    pltpu.make_async_copy(src_hbm.at[pl.ds(0, _EPT)], srcbuf, lsem).wait()
    pltpu.make_async_copy(src_hbm.at[pl.ds(0, _EPT)], kidxbuf, lsem).wait()
    pltpu.make_async_copy(dst_hbm.at[pl.ds(0, _NCH)], dstbuf, lsem).wait()

    for t in range(_RPT // _ZR):
        pltpu.async_copy(zbuf, acc.at[pl.ds(s * _RPT + t * _ZR, _ZR)], lsem)

    def build_idx(i, carry):
        o = i * _L
        kv = kidxbuf[pl.ds(o, _L)]
        sv = srcbuf[pl.ds(o, _L)]
        gidx[pl.ds(o, _L)] = (
            lax.shift_right_logical(kv, 2) * (4 * _N)
            + sv * 4 + lax.bitwise_and(kv, 3))
        return carry
    lax.fori_loop(0, _EPT // _L, build_idx, 0)

    for b in range(_NBUF):
        gather(b, b)

    for t in range(_RPT // _ZR):
        pltpu.make_async_copy(ht_hbm.at[pl.ds(0, _ZR)], zbuf, lsem).wait()
    plsc.subcore_barrier()

    def chunk_round(jj, carry):
        j = jj * _NBUF
        for b in range(_NBUF):
            wait_gather(b)
            pltpu.sync_copy(rows.at[b], acc.at[dstbuf.at[j + b]], add=True)

            @pl.when(j + b + _NBUF < _NCH)
            def _():
                gather(j + b + _NBUF, b)
        return carry
    lax.fori_loop(0, _NCH // _NBUF, chunk_round, 0)
    plsc.subcore_barrier()

    pltpu.sync_copy(acc.at[pl.ds(s * _RPT, _RPT)],
                    out_hbm.at[pl.ds(c * _NACC + s * _RPT, _RPT)])


@jax.jit
def kernel(x, W1, b1, W2, b2, W3, b3, edge_src, edge_dst, edge_kidx):
    npad = _EPAD - _E
    spread = jnp.arange(npad, dtype=jnp.int32)
    src_p = jnp.concatenate([edge_src, spread % _N])
    kidx_p = jnp.concatenate([edge_kidx, jnp.zeros((npad,), jnp.int32)])
    dst_p = jnp.concatenate(
        [edge_dst, _N + spread % (_NACC - _N)]).reshape(-1, _CHUNK)

    bn1 = 512
    grid1 = pl.cdiv(_N, bn1)
    w2all = jnp.transpose(W2, (1, 0, 2)).reshape(_CB, _K * _CB)
    w2pad = jnp.concatenate(
        [w2all, jnp.zeros((_CB, _NT * 128 - _K * _CB), jnp.float32)], axis=1)
    ht = pl.pallas_call(
        _tc1_body,
        grid=(grid1,),
        in_specs=[
            pl.BlockSpec((bn1, _NIN), lambda i: (i, 0)),
            pl.BlockSpec((_NIN, _CB), lambda i: (0, 0)),
            pl.BlockSpec((1, _CB), lambda i: (0, 0)),
            pl.BlockSpec((_CB, _NT * 128), lambda i: (0, 0)),
        ],
        out_specs=pl.BlockSpec((_NT, bn1, 128), lambda i: (0, i, 0)),
        out_shape=jax.ShapeDtypeStruct((_NT, _N, 128), jnp.float32),
    )(x, W1, b1.reshape(1, _CB), w2pad)
    ht_rows = ht.reshape(_NT * _N * 4, _CB)

    mesh = plsc.VectorSubcoreMesh(core_axis_name="c", subcore_axis_name="s")
    partial = pl.kernel(
        _sc_body,
        out_type=jax.ShapeDtypeStruct((_NC * _NACC, _CB), jnp.float32),
        mesh=mesh,
        compiler_params=pltpu.CompilerParams(use_tc_tiling_on_sc=False),
        scratch_types=[
            pltpu.VMEM((_EPT,), jnp.int32),
            pltpu.VMEM((_EPT,), jnp.int32),
            pltpu.VMEM((_EPT,), jnp.int32),
            pltpu.VMEM((_NCH, _CHUNK), jnp.int32),
            pltpu.VMEM((_NBUF, _CHUNK, _CB), jnp.float32),
            pltpu.VMEM((_ZR, _CB), jnp.float32),
            pltpu.VMEM_SHARED((_NACC, _CB), jnp.float32),
        ] + [pltpu.SemaphoreType.DMA] * (_NBUF + 1),
    )(ht_rows, src_p, kidx_p, dst_p)
    psum = partial.reshape(_NC, _NACC // 4, 4 * _CB)
    x4 = x.reshape(_N // 4, 4 * _NIN)

    bn2 = 512
    grid2 = pl.cdiv(_N // 4, bn2)
    out = pl.pallas_call(
        _tc2_body,
        grid=(grid2,),
        in_specs=[
            pl.BlockSpec((_NC, bn2, 4 * _CB), lambda i: (0, i, 0)),
            pl.BlockSpec((bn2, 4 * _NIN), lambda i: (i, 0)),
            pl.BlockSpec((1, _CB), lambda i: (0, 0)),
            pl.BlockSpec((_CB, _NOUT), lambda i: (0, 0)),
            pl.BlockSpec((1, _NOUT), lambda i: (0, 0)),
        ],
        out_specs=pl.BlockSpec((4 * bn2, _NOUT), lambda i: (i, 0)),
        out_shape=jax.ShapeDtypeStruct((_N, _NOUT), jnp.float32),
    )(psum, x4, b2.reshape(1, _CB), W3, b3.reshape(1, _NOUT))
    return out


# speedup vs baseline: 1.9073x; 1.0790x over previous
import functools

import jax
import jax.numpy as jnp
from jax import lax
from jax.experimental import pallas as pl
from jax.experimental.pallas import tpu as pltpu
from jax.experimental.pallas import tpu_sc as plsc

_N = 10000
_E = 160000
_NIN = 128
_NOUT = 128
_CB = 32
_K = 27

_NC = 2
_NS = 16
_L = 16

_EPAD = 163840
_CHUNK = 128
_NCH = 40
_EPT = _NCH * _CHUNK
_NBUF = 8
_NACC = 10240
_JUNK = 10200
_RPT = _NACC // _NS
_ZR = 64
_NT = 7


def _tc1_body(x_ref, w1_ref, b1_ref, w2_ref, ht_ref):
    h = jnp.dot(x_ref[...], w1_ref[...], preferred_element_type=jnp.float32)
    h = jnp.maximum(h + b1_ref[...], 0.0)
    for t in range(_NT):
        ht_ref[t] = jnp.dot(h, w2_ref[:, pl.ds(t * 128, 128)],
                            preferred_element_type=jnp.float32)


def _tc2_body(p_ref, x_ref, b2_ref, w3_ref, b3_ref, o_ref):
    bn = p_ref.shape[1]
    slabs = []
    for q in range(4):
        qs = pl.ds(q * _CB, _CB)
        h2 = jnp.maximum(p_ref[0, :, qs] + p_ref[1, :, qs] + b2_ref[...], 0.0)
        h3 = jnp.dot(h2, w3_ref[...], preferred_element_type=jnp.float32)
        qo = pl.ds(q * _NOUT, _NOUT)
        slabs.append(jnp.maximum(
            (h3 + b3_ref[...] + x_ref[:, qo]) * 0.5, 0.0))
    y = jnp.concatenate(slabs, axis=1)
    o_ref[...] = y.reshape(4 * bn, _NOUT)


def _sc_body(ht_hbm, src_hbm, kidx_hbm, dst_hbm, out_hbm,
             srcbuf, kidxbuf, gidx, dstbuf, rows, zbuf, acc,
             sg0, sg1, sg2, sg3, sg4, sg5, sg6, sg7, lsem):
    sems = (sg0, sg1, sg2, sg3, sg4, sg5, sg6, sg7)
    c = lax.axis_index("c")
    s = lax.axis_index("s")
    wid = c * _NS + s
    ebase = wid * _EPT

    def gather(j, b):
        pltpu.async_copy(ht_hbm.at[gidx.at[pl.ds(j * _CHUNK, _CHUNK)]],
                         rows.at[b], sems[b])

    def wait_gather(b):
        pltpu.make_async_copy(ht_hbm.at[pl.ds(0, _CHUNK)], rows.at[b],
                              sems[b]).wait()

    pltpu.async_copy(src_hbm.at[pl.ds(ebase, _EPT)], srcbuf, lsem)
    pltpu.async_copy(kidx_hbm.at[pl.ds(ebase, _EPT)], kidxbuf, lsem)
    pltpu.async_copy(dst_hbm.at[pl.ds(wid * _NCH, _NCH)], dstbuf, lsem)

    z16 = jnp.zeros((_L,), jnp.float32)

    def zero_zbuf(r, carry):
        zbuf[r, pl.ds(0, _L)] = z16
        zbuf[r, pl.ds(_L, _L)] = z16
        return carry
    lax.fori_loop(0, _ZR, zero_zbuf, 0)

    pltpu.make_async_copy(src_hbm.at[pl.ds(0, _EPT)], srcbuf, lsem).wait()
    pltpu.make_async_copy(src_hbm.at[pl.ds(0, _EPT)], kidxbuf, lsem).wait()
    pltpu.make_async_copy(dst_hbm.at[pl.ds(0, _NCH)], dstbuf, lsem).wait()

    for t in range(_RPT // _ZR):
        pltpu.async_copy(zbuf, acc.at[pl.ds(s * _RPT + t * _ZR, _ZR)], lsem)

    def build_idx(i, carry):
        o = i * _L
        kv = kidxbuf[pl.ds(o, _L)]
        sv = srcbuf[pl.ds(o, _L)]
        gidx[pl.ds(o, _L)] = (
            lax.shift_right_logical(kv, 2) * (4 * _N)
            + sv * 4 + lax.bitwise_and(kv, 3))
        return carry
    lax.fori_loop(0, _EPT // _L, build_idx, 0)

    for b in range(_NBUF):
        gather(b, b)

    for t in range(_RPT // _ZR):
        pltpu.make_async_copy(ht_hbm.at[pl.ds(0, _ZR)], zbuf, lsem).wait()
    plsc.subcore_barrier()

    def chunk_round(jj, carry):
        j = jj * _NBUF
        for b in range(_NBUF):
            wait_gather(b)
            pltpu.sync_copy(rows.at[b], acc.at[dstbuf.at[j + b]], add=True)

            @pl.when(j + b + _NBUF < _NCH)
            def _():
                gather(j + b + _NBUF, b)
        return carry
    lax.fori_loop(0, _NCH // _NBUF, chunk_round, 0)
    plsc.subcore_barrier()

    pltpu.sync_copy(acc.at[pl.ds(s * _RPT, _RPT)],
                    out_hbm.at[pl.ds(c * _NACC + s * _RPT, _RPT)])


@jax.jit
def kernel(x, W1, b1, W2, b2, W3, b3, edge_src, edge_dst, edge_kidx):
    npad = _EPAD - _E
    spread = jnp.arange(npad, dtype=jnp.int32)
    src_p = jnp.concatenate([edge_src, spread % _N])
    kidx_p = jnp.concatenate([edge_kidx, jnp.zeros((npad,), jnp.int32)])
    dst_p = jnp.concatenate(
        [edge_dst, _N + spread % (_NACC - _N)]).reshape(-1, _CHUNK)

    bn1 = 1024
    grid1 = pl.cdiv(_N, bn1)
    w2all = jnp.transpose(W2, (1, 0, 2)).reshape(_CB, _K * _CB)
    w2pad = jnp.concatenate(
        [w2all, jnp.zeros((_CB, _NT * 128 - _K * _CB), jnp.float32)], axis=1)
    ht = pl.pallas_call(
        _tc1_body,
        grid=(grid1,),
        in_specs=[
            pl.BlockSpec((bn1, _NIN), lambda i: (i, 0)),
            pl.BlockSpec((_NIN, _CB), lambda i: (0, 0)),
            pl.BlockSpec((1, _CB), lambda i: (0, 0)),
            pl.BlockSpec((_CB, _NT * 128), lambda i: (0, 0)),
        ],
        out_specs=pl.BlockSpec((_NT, bn1, 128), lambda i: (0, i, 0)),
        out_shape=jax.ShapeDtypeStruct((_NT, _N, 128), jnp.float32),
    )(x, W1, b1.reshape(1, _CB), w2pad)
    ht_rows = ht.reshape(_NT * _N * 4, _CB)

    mesh = plsc.VectorSubcoreMesh(core_axis_name="c", subcore_axis_name="s")
    partial = pl.kernel(
        _sc_body,
        out_type=jax.ShapeDtypeStruct((_NC * _NACC, _CB), jnp.float32),
        mesh=mesh,
        compiler_params=pltpu.CompilerParams(use_tc_tiling_on_sc=False),
        scratch_types=[
            pltpu.VMEM((_EPT,), jnp.int32),
            pltpu.VMEM((_EPT,), jnp.int32),
            pltpu.VMEM((_EPT,), jnp.int32),
            pltpu.VMEM((_NCH, _CHUNK), jnp.int32),
            pltpu.VMEM((_NBUF, _CHUNK, _CB), jnp.float32),
            pltpu.VMEM((_ZR, _CB), jnp.float32),
            pltpu.VMEM_SHARED((_NACC, _CB), jnp.float32),
        ] + [pltpu.SemaphoreType.DMA] * (_NBUF + 1),
    )(ht_rows, src_p, kidx_p, dst_p)
    psum = partial.reshape(_NC, _NACC // 4, 4 * _CB)
    x4 = x.reshape(_N // 4, 4 * _NIN)

    bn2 = 512
    grid2 = pl.cdiv(_N // 4, bn2)
    out = pl.pallas_call(
        _tc2_body,
        grid=(grid2,),
        in_specs=[
            pl.BlockSpec((_NC, bn2, 4 * _CB), lambda i: (0, i, 0)),
            pl.BlockSpec((bn2, 4 * _NIN), lambda i: (i, 0)),
            pl.BlockSpec((1, _CB), lambda i: (0, 0)),
            pl.BlockSpec((_CB, _NOUT), lambda i: (0, 0)),
            pl.BlockSpec((1, _NOUT), lambda i: (0, 0)),
        ],
        out_specs=pl.BlockSpec((4 * bn2, _NOUT), lambda i: (i, 0)),
        out_shape=jax.ShapeDtypeStruct((_N, _NOUT), jnp.float32),
    )(psum, x4, b2.reshape(1, _CB), W3, b3.reshape(1, _NOUT))
    return out


# speedup vs baseline: 1.9775x; 1.0368x over previous
import functools

import jax
import jax.numpy as jnp
from jax import lax
from jax.experimental import pallas as pl
from jax.experimental.pallas import tpu as pltpu
from jax.experimental.pallas import tpu_sc as plsc

_N = 10000
_E = 160000
_NIN = 128
_NOUT = 128
_CB = 32
_K = 27

_NC = 2
_NS = 16
_L = 16

_EPAD = 163840
_CHUNK = 128
_NCH = 40
_EPT = _NCH * _CHUNK
_NBUF = 8
_NACC = 10240
_JUNK = 10200
_RPT = _NACC // _NS
_ZR = 64
_NT = 7


def _tc1_body(x_ref, w1_ref, b1_ref, w2_ref, ht_ref):
    h = jnp.dot(x_ref[...], w1_ref[...], preferred_element_type=jnp.float32)
    h = jnp.maximum(h + b1_ref[...], 0.0)
    for t in range(_NT):
        ht_ref[t] = jnp.dot(h, w2_ref[:, pl.ds(t * 128, 128)],
                            preferred_element_type=jnp.float32)


def _tc2_body(p_ref, x_ref, b2_ref, w3_ref, b3_ref, o_ref):
    bn = p_ref.shape[1]
    slabs = []
    for q in range(4):
        qs = pl.ds(q * _CB, _CB)
        h2 = jnp.maximum(p_ref[0, :, qs] + p_ref[1, :, qs] + b2_ref[...], 0.0)
        h3 = jnp.dot(h2, w3_ref[...], preferred_element_type=jnp.float32)
        qo = pl.ds(q * _NOUT, _NOUT)
        slabs.append(jnp.maximum(
            (h3 + b3_ref[...] + x_ref[:, qo]) * 0.5, 0.0))
    y = jnp.concatenate(slabs, axis=1)
    o_ref[...] = y.reshape(4 * bn, _NOUT)


def _sc_body(ht_hbm, src_hbm, kidx_hbm, dst_hbm, out_hbm,
             srcbuf, kidxbuf, gidx, dstbuf, rows, zbuf, acc,
             sg0, sg1, sg2, sg3, sg4, sg5, sg6, sg7, lsem):
    sems = (sg0, sg1, sg2, sg3, sg4, sg5, sg6, sg7)
    c = lax.axis_index("c")
    s = lax.axis_index("s")
    wid = c * _NS + s
    ebase = wid * _EPT

    def gather(j, b):
        pltpu.async_copy(ht_hbm.at[gidx.at[pl.ds(j * _CHUNK, _CHUNK)]],
                         rows.at[b], sems[b])

    def wait_gather(b):
        pltpu.make_async_copy(ht_hbm.at[pl.ds(0, _CHUNK)], rows.at[b],
                              sems[b]).wait()

    pltpu.async_copy(src_hbm.at[pl.ds(ebase, _EPT)], srcbuf, lsem)
    pltpu.async_copy(kidx_hbm.at[pl.ds(ebase, _EPT)], kidxbuf, lsem)
    pltpu.async_copy(dst_hbm.at[pl.ds(wid * _NCH, _NCH)], dstbuf, lsem)

    z16 = jnp.zeros((_L,), jnp.float32)

    def zero_zbuf(r, carry):
        zbuf[r, pl.ds(0, _L)] = z16
        zbuf[r, pl.ds(_L, _L)] = z16
        return carry
    lax.fori_loop(0, _ZR, zero_zbuf, 0)

    pltpu.make_async_copy(src_hbm.at[pl.ds(0, _EPT)], srcbuf, lsem).wait()
    pltpu.make_async_copy(src_hbm.at[pl.ds(0, _EPT)], kidxbuf, lsem).wait()
    pltpu.make_async_copy(dst_hbm.at[pl.ds(0, _NCH)], dstbuf, lsem).wait()

    for t in range(_RPT // _ZR):
        pltpu.async_copy(zbuf, acc.at[pl.ds(s * _RPT + t * _ZR, _ZR)], lsem)

    def build_idx(i, carry):
        o = i * _L
        kv = kidxbuf[pl.ds(o, _L)]
        sv = srcbuf[pl.ds(o, _L)]
        gidx[pl.ds(o, _L)] = (
            lax.shift_right_logical(kv, 2) * (4 * _N)
            + sv * 4 + lax.bitwise_and(kv, 3))
        return carry
    lax.fori_loop(0, _EPT // _L, build_idx, 0)

    for b in range(_NBUF):
        gather(b, b)

    for t in range(_RPT // _ZR):
        pltpu.make_async_copy(ht_hbm.at[pl.ds(0, _ZR)], zbuf, lsem).wait()
    plsc.subcore_barrier()

    def chunk_round(jj, carry):
        j = jj * _NBUF
        for b in range(_NBUF):
            wait_gather(b)
            pltpu.sync_copy(rows.at[b], acc.at[dstbuf.at[j + b]], add=True)

            @pl.when(j + b + _NBUF < _NCH)
            def _():
                gather(j + b + _NBUF, b)
        return carry
    lax.fori_loop(0, _NCH // _NBUF, chunk_round, 0)
    plsc.subcore_barrier()

    pltpu.sync_copy(acc.at[pl.ds(s * _RPT, _RPT)],
                    out_hbm.at[pl.ds(c * _NACC + s * _RPT, _RPT)])


@jax.jit
def kernel(x, W1, b1, W2, b2, W3, b3, edge_src, edge_dst, edge_kidx):
    npad = _EPAD - _E
    spread = jnp.arange(npad, dtype=jnp.int32)
    src_p = jnp.concatenate([edge_src, spread % _N])
    kidx_p = jnp.concatenate([edge_kidx, jnp.zeros((npad,), jnp.int32)])
    dst_p = jnp.concatenate(
        [edge_dst, _N + spread % (_NACC - _N)]).reshape(-1, _CHUNK)

    bn1 = 2048
    grid1 = pl.cdiv(_N, bn1)
    w2all = jnp.transpose(W2, (1, 0, 2)).reshape(_CB, _K * _CB)
    w2pad = jnp.concatenate(
        [w2all, jnp.zeros((_CB, _NT * 128 - _K * _CB), jnp.float32)], axis=1)
    ht = pl.pallas_call(
        _tc1_body,
        grid=(grid1,),
        in_specs=[
            pl.BlockSpec((bn1, _NIN), lambda i: (i, 0)),
            pl.BlockSpec((_NIN, _CB), lambda i: (0, 0)),
            pl.BlockSpec((1, _CB), lambda i: (0, 0)),
            pl.BlockSpec((_CB, _NT * 128), lambda i: (0, 0)),
        ],
        out_specs=pl.BlockSpec((_NT, bn1, 128), lambda i: (0, i, 0)),
        out_shape=jax.ShapeDtypeStruct((_NT, _N, 128), jnp.float32),
    )(x, W1, b1.reshape(1, _CB), w2pad)
    ht_rows = ht.reshape(_NT * _N * 4, _CB)

    mesh = plsc.VectorSubcoreMesh(core_axis_name="c", subcore_axis_name="s")
    partial = pl.kernel(
        _sc_body,
        out_type=jax.ShapeDtypeStruct((_NC * _NACC, _CB), jnp.float32),
        mesh=mesh,
        compiler_params=pltpu.CompilerParams(use_tc_tiling_on_sc=False),
        scratch_types=[
            pltpu.VMEM((_EPT,), jnp.int32),
            pltpu.VMEM((_EPT,), jnp.int32),
            pltpu.VMEM((_EPT,), jnp.int32),
            pltpu.VMEM((_NCH, _CHUNK), jnp.int32),
            pltpu.VMEM((_NBUF, _CHUNK, _CB), jnp.float32),
            pltpu.VMEM((_ZR, _CB), jnp.float32),
            pltpu.VMEM_SHARED((_NACC, _CB), jnp.float32),
        ] + [pltpu.SemaphoreType.DMA] * (_NBUF + 1),
    )(ht_rows, src_p, kidx_p, dst_p)
    psum = partial.reshape(_NC, _NACC // 4, 4 * _CB)
    x4 = x.reshape(_N // 4, 4 * _NIN)

    bn2 = 512
    grid2 = pl.cdiv(_N // 4, bn2)
    out = pl.pallas_call(
        _tc2_body,
        grid=(grid2,),
        in_specs=[
            pl.BlockSpec((_NC, bn2, 4 * _CB), lambda i: (0, i, 0)),
            pl.BlockSpec((bn2, 4 * _NIN), lambda i: (i, 0)),
            pl.BlockSpec((1, _CB), lambda i: (0, 0)),
            pl.BlockSpec((_CB, _NOUT), lambda i: (0, 0)),
            pl.BlockSpec((1, _NOUT), lambda i: (0, 0)),
        ],
        out_specs=pl.BlockSpec((4 * bn2, _NOUT), lambda i: (i, 0)),
        out_shape=jax.ShapeDtypeStruct((_N, _NOUT), jnp.float32),
    )(psum, x4, b2.reshape(1, _CB), W3, b3.reshape(1, _NOUT))
    return out
